# baseline (speedup 1.0000x reference)
import jax
import jax.numpy as jnp
from jax.experimental import pallas as pl
from jax.experimental.pallas import tpu as pltpu

_ROWS = 16384


def _dma_kernel(x_hbm, o_hbm):
    def body(sem):
        copy = pltpu.make_async_copy(x_hbm.at[:, pl.ds(15, 3), :], o_hbm, sem)
        copy.start()
        copy.wait()

    pl.run_scoped(body, pltpu.SemaphoreType.DMA)


def kernel(input):
    return pl.pallas_call(
        _dma_kernel,
        in_specs=[pl.BlockSpec(memory_space=pltpu.MemorySpace.HBM)],
        out_specs=pl.BlockSpec(memory_space=pltpu.MemorySpace.HBM),
        out_shape=jax.ShapeDtypeStruct((_ROWS, 3, 128), input.dtype),
    )(input)


# speedup vs baseline: 10.6752x; 10.6752x over previous
import jax
import jax.numpy as jnp
from jax.experimental import pallas as pl

_ROWS = 16384
_BLOCK_ROWS = 2048


def _sel_kernel(a_ref, b_ref, o_ref):
    o_ref[:, 0, :] = a_ref[:, 0, 7, :]
    o_ref[:, 1:3, :] = b_ref[:, 0, 0:2, :]


def kernel(input):
    n = _ROWS // _BLOCK_ROWS
    x = input.reshape(_ROWS, 4, 8, 128)
    return pl.pallas_call(
        _sel_kernel,
        grid=(n,),
        in_specs=[
            pl.BlockSpec((_BLOCK_ROWS, 1, 8, 128), lambda i: (i, 1, 0, 0)),
            pl.BlockSpec((_BLOCK_ROWS, 1, 8, 128), lambda i: (i, 2, 0, 0)),
        ],
        out_specs=pl.BlockSpec((_BLOCK_ROWS, 3, 128), lambda i: (i, 0, 0)),
        out_shape=jax.ShapeDtypeStruct((_ROWS, 3, 128), input.dtype),
    )(x, x)


# speedup vs baseline: 10.7850x; 1.0103x over previous
import jax
import jax.numpy as jnp
from jax.experimental import pallas as pl

_ROWS = 16384
_BLOCK_ROWS = 1024


def _sel_kernel(a_ref, b_ref, o_ref):
    o_ref[:, 0, :] = a_ref[:, 0, 7, :]
    o_ref[:, 1:3, :] = b_ref[:, 0, 0:2, :]


def kernel(input):
    n = _ROWS // _BLOCK_ROWS
    x = input.reshape(_ROWS, 4, 8, 128)
    return pl.pallas_call(
        _sel_kernel,
        grid=(n,),
        in_specs=[
            pl.BlockSpec((_BLOCK_ROWS, 1, 8, 128), lambda i: (i, 1, 0, 0)),
            pl.BlockSpec((_BLOCK_ROWS, 1, 8, 128), lambda i: (i, 2, 0, 0)),
        ],
        out_specs=pl.BlockSpec((_BLOCK_ROWS, 3, 128), lambda i: (i, 0, 0)),
        out_shape=jax.ShapeDtypeStruct((_ROWS, 3, 128), input.dtype),
    )(x, x)
